# Initial kernel scaffold; baseline (speedup 1.0000x reference)
import jax
import jax.numpy as jnp
from jax.experimental import pallas as pl


def kernel(x, edge_index, W1, g1, b1, W2, g2, b2, Wl, bl, Ws, bs):
    raise NotImplementedError("write your pallas kernel here")


# speedup vs baseline: 10.4772x; 10.4772x over previous
import functools

import jax
import jax.numpy as jnp
from jax import lax
from jax.experimental import pallas as pl
from jax.experimental.pallas import tpu as pltpu
from jax.experimental.pallas import tpu_sc as plsc

N = 10000
E = 320000
D_IN = 128
H = 256
D_OUT = 64
EPS = 1e-07
TOTAL_COUNT = 10000.0

NC = 2
NS = 16
CS = 80
NP = 10240
RPT = NP // NS
RB = 2000

_mesh = plsc.VectorSubcoreMesh(core_axis_name="c", subcore_axis_name="s")


@functools.partial(
    pl.kernel,
    out_type=jax.ShapeDtypeStruct((NC, NP, 16), jnp.float32),
    mesh=_mesh,
    scratch_types=[
        pltpu.VMEM((CS,), jnp.int32),
        pltpu.VMEM((CS, 16), jnp.float32),
        pltpu.VMEM_SHARED((NP, 16), jnp.float32),
    ],
)
def _deg_kernel(dsts, ones, zeros16, out, dst_v, ones_v, acc):
    c = lax.axis_index("c")
    s = lax.axis_index("s")
    pltpu.sync_copy(ones, ones_v)
    pltpu.sync_copy(zeros16.at[pl.ds(s * RPT, RPT)], acc.at[pl.ds(s * RPT, RPT)])
    plsc.subcore_barrier()
    ept = (E // NC) // NS
    base = c * (E // NC) + s * ept

    def step(i, carry):
        off = base + i * CS
        pltpu.sync_copy(dsts.at[pl.ds(off, CS)], dst_v)
        pltpu.sync_copy(ones_v, acc.at[dst_v], add=True)
        return carry

    lax.fori_loop(0, ept // CS, step, 0)
    plsc.subcore_barrier()
    pltpu.sync_copy(acc.at[pl.ds(s * RPT, RPT)], out.at[c, pl.ds(s * RPT, RPT)])


def _make_agg(epc):
    ept = epc // NS

    @functools.partial(
        pl.kernel,
        out_type=jax.ShapeDtypeStruct((NC, NP, 128), jnp.float32),
        mesh=_mesh,
        scratch_types=[
            pltpu.VMEM((CS,), jnp.int32),
            pltpu.VMEM((CS,), jnp.int32),
            pltpu.VMEM((CS, 128), jnp.float32),
            pltpu.VMEM_SHARED((NP, 128), jnp.float32),
            pltpu.SemaphoreType.DMA,
        ],
    )
    def _agg(table, idxs, dsts, zeros, out, idx_v, dst_v, rows_v, acc, sem):
        c = lax.axis_index("c")
        s = lax.axis_index("s")
        pltpu.sync_copy(zeros.at[pl.ds(s * RPT, RPT)], acc.at[pl.ds(s * RPT, RPT)])
        plsc.subcore_barrier()
        base = c * epc + s * ept

        def step(i, carry):
            off = base + i * CS
            pltpu.sync_copy(idxs.at[pl.ds(off, CS)], idx_v)
            pltpu.sync_copy(dsts.at[pl.ds(off, CS)], dst_v)
            pltpu.async_copy(table.at[idx_v], rows_v, sem).wait()
            pltpu.sync_copy(rows_v, acc.at[dst_v], add=True)
            return carry

        lax.fori_loop(0, ept // CS, step, 0)
        plsc.subcore_barrier()
        pltpu.sync_copy(acc.at[pl.ds(s * RPT, RPT)], out.at[c, pl.ds(s * RPT, RPT)])

    return _agg


_agg_half = _make_agg(E // NC)
_agg_full = _make_agg(E)


def _pre_body(x_ref, l_ref, ptr_ref):
    x = x_ref[...]
    s = jnp.sum(x, axis=1, keepdims=True)
    l_ref[...] = s
    ptr_ref[...] = jnp.log1p(x * (TOTAL_COUNT / s))


def _pre(x):
    return pl.pallas_call(
        _pre_body,
        grid=(N // RB,),
        in_specs=[pl.BlockSpec((RB, D_IN), lambda i: (i, 0))],
        out_specs=[
            pl.BlockSpec((RB, 1), lambda i: (i, 0)),
            pl.BlockSpec((RB, D_IN), lambda i: (i, 0)),
        ],
        out_shape=[
            jax.ShapeDtypeStruct((N, 1), jnp.float32),
            jax.ShapeDtypeStruct((N, D_IN), jnp.float32),
        ],
    )(x)


def _scale_body(degp_ref, ptr_ref, d_ref, t1_ref):
    deg = 1.0 + degp_ref[0, :, 0:1] + degp_ref[1, :, 0:1]
    d = lax.rsqrt(deg)
    d_ref[...] = d
    t1_ref[...] = d * ptr_ref[...]


def _scale(degp, ptr):
    return pl.pallas_call(
        _scale_body,
        grid=(N // RB,),
        in_specs=[
            pl.BlockSpec((2, RB, 16), lambda i: (0, i, 0)),
            pl.BlockSpec((RB, D_IN), lambda i: (i, 0)),
        ],
        out_specs=[
            pl.BlockSpec((RB, 1), lambda i: (i, 0)),
            pl.BlockSpec((RB, D_IN), lambda i: (i, 0)),
        ],
        out_shape=[
            jax.ShapeDtypeStruct((N, 1), jnp.float32),
            jax.ShapeDtypeStruct((N, D_IN), jnp.float32),
        ],
    )(degp, ptr)


def _l1_body(p_ref, t1_ref, d_ref, w_ref, b_ref, out_ref):
    d = d_ref[...]
    agg = d * (p_ref[0] + p_ref[1] + t1_ref[...])
    z = jnp.dot(agg, w_ref[...], preferred_element_type=jnp.float32) + b_ref[...]
    h = jnp.where(z >= 0, z, 0.2 * z)
    out_ref[0] = d * h


def _l1(p, t1, d, w1eff, b1):
    return pl.pallas_call(
        _l1_body,
        grid=(2, N // RB),
        in_specs=[
            pl.BlockSpec((2, RB, 128), lambda h, j: (0, j, 0)),
            pl.BlockSpec((RB, D_IN), lambda h, j: (j, 0)),
            pl.BlockSpec((RB, 1), lambda h, j: (j, 0)),
            pl.BlockSpec((D_IN, 128), lambda h, j: (0, h)),
            pl.BlockSpec((1, 128), lambda h, j: (0, h)),
        ],
        out_specs=pl.BlockSpec((1, RB, 128), lambda h, j: (h, j, 0)),
        out_shape=jax.ShapeDtypeStruct((2, N, 128), jnp.float32),
    )(p, t1, d, w1eff, b1)


def _l2_body(q_ref, t2_ref, d_ref, w2_ref, b2_ref, wl_ref, bl_ref, ws_ref,
             bs_ref, loc_ref, std_ref):
    d = d_ref[...]
    a_lo = d * (q_ref[0] + t2_ref[0])
    a_hi = d * (q_ref[1] + t2_ref[1])
    w2 = w2_ref[...]
    z = (jnp.dot(a_lo, w2[:128], preferred_element_type=jnp.float32)
         + jnp.dot(a_hi, w2[128:], preferred_element_type=jnp.float32)
         + b2_ref[...])
    h = jnp.where(z >= 0, z, 0.2 * z)
    loc_ref[...] = (jnp.dot(h, wl_ref[...], preferred_element_type=jnp.float32)
                    + bl_ref[...])
    t = jnp.dot(h, ws_ref[...], preferred_element_type=jnp.float32) + bs_ref[...]
    std_ref[...] = jnp.maximum(t, 0.0) + jnp.log1p(jnp.exp(-jnp.abs(t))) + EPS


def _l2(q, t2, d, w2eff, b2, wl, bl, ws, bs):
    return pl.pallas_call(
        _l2_body,
        grid=(N // RB,),
        in_specs=[
            pl.BlockSpec((2, RB, 128), lambda j: (0, j, 0)),
            pl.BlockSpec((2, RB, 128), lambda j: (0, j, 0)),
            pl.BlockSpec((RB, 1), lambda j: (j, 0)),
            pl.BlockSpec((H, H), lambda j: (0, 0)),
            pl.BlockSpec((1, H), lambda j: (0, 0)),
            pl.BlockSpec((H, D_OUT), lambda j: (0, 0)),
            pl.BlockSpec((1, D_OUT), lambda j: (0, 0)),
            pl.BlockSpec((H, D_OUT), lambda j: (0, 0)),
            pl.BlockSpec((1, D_OUT), lambda j: (0, 0)),
        ],
        out_specs=[
            pl.BlockSpec((RB, D_OUT), lambda j: (j, 0)),
            pl.BlockSpec((RB, D_OUT), lambda j: (j, 0)),
        ],
        out_shape=[
            jax.ShapeDtypeStruct((N, D_OUT), jnp.float32),
            jax.ShapeDtypeStruct((N, D_OUT), jnp.float32),
        ],
    )(q, t2, d, w2eff, b2, wl, bl, ws, bs)


def kernel(x, edge_index, W1, g1, b1, W2, g2, b2, Wl, bl, Ws, bs):
    src = edge_index[0]
    dst = edge_index[1]

    zeros16 = jnp.zeros((NP, 16), jnp.float32)
    zeros128 = jnp.zeros((NP, 128), jnp.float32)
    ones = jnp.ones((CS, 16), jnp.float32)

    degp = _deg_kernel(dst, ones, zeros16)
    l, ptr = _pre(x)
    d, t1 = _scale(degp, ptr)

    p = _agg_half(t1, src, dst, zeros128)
    bn1 = (1.0 + 1e-5) ** -0.5
    w1eff = W1 * (g1 * bn1)[None, :]
    t2 = _l1(p, t1, d, w1eff, b1.reshape(1, H))

    idx2 = jnp.concatenate([src, src + N])
    dst2 = jnp.concatenate([dst, dst])
    q = _agg_full(t2.reshape(2 * N, 128), idx2, dst2, zeros128)

    w2eff = W2 * (g2 * bn1)[None, :]
    loc, std = _l2(q, t2, d, w2eff, b2.reshape(1, H),
                   Wl, bl.reshape(1, D_OUT), Ws, bs.reshape(1, D_OUT))
    return (loc, std, l)


# speedup vs baseline: 25.1590x; 2.4013x over previous
import functools

import jax
import jax.numpy as jnp
from jax import lax
from jax.experimental import pallas as pl
from jax.experimental.pallas import tpu as pltpu
from jax.experimental.pallas import tpu_sc as plsc

N = 10000
E = 320000
D_IN = 128
H = 256
D_OUT = 64
EPS = 1e-07
TOTAL_COUNT = 10000.0

NC = 2
NS = 16
CS = 80
CSA = 40
NP = 10240
RPT = NP // NS
RB = 2000

_mesh = plsc.VectorSubcoreMesh(core_axis_name="c", subcore_axis_name="s")


NB = 5


@functools.partial(
    pl.kernel,
    out_type=jax.ShapeDtypeStruct((NC, NP, 16), jnp.float32),
    mesh=_mesh,
    scratch_types=[
        [pltpu.VMEM((CS,), jnp.int32) for _ in range(NB)],
        pltpu.VMEM((CS, 16), jnp.float32),
        pltpu.VMEM_SHARED((NP, 16), jnp.float32),
        [pltpu.SemaphoreType.DMA for _ in range(NB)],
        [pltpu.SemaphoreType.DMA for _ in range(NB)],
    ],
)
def _deg_kernel(dsts, ones, zeros16, out, dst_v, ones_v, acc, isem, ssem):
    c = lax.axis_index("c")
    s = lax.axis_index("s")
    ept = (E // NC) // NS
    base = c * (E // NC) + s * ept
    nsteps = ept // CS
    rounds = nsteps // NB

    for b in range(NB):
        pltpu.async_copy(dsts.at[pl.ds(base + b * CS, CS)], dst_v[b], isem[b])
    pltpu.sync_copy(ones, ones_v)
    pltpu.sync_copy(zeros16.at[pl.ds(s * RPT, RPT)], acc.at[pl.ds(s * RPT, RPT)])
    plsc.subcore_barrier()

    def round_(r, carry):
        for b in range(NB):
            pltpu.make_async_copy(dsts.at[pl.ds(base, CS)], dst_v[b], isem[b]).wait()
            pltpu.async_copy(ones_v, acc.at[dst_v[b]], ssem[b], add=True)
        for b in range(NB):
            @pl.when(r != rounds - 1)
            def _():
                pltpu.make_async_copy(ones_v, acc.at[dst_v[b]], ssem[b]).wait()
                off = base + ((r + 1) * NB + b) * CS
                pltpu.async_copy(dsts.at[pl.ds(off, CS)], dst_v[b], isem[b])
        return carry

    lax.fori_loop(0, rounds, round_, 0)
    for b in range(NB):
        pltpu.make_async_copy(ones_v, acc.at[dst_v[b]], ssem[b]).wait()
    plsc.subcore_barrier()
    pltpu.sync_copy(acc.at[pl.ds(s * RPT, RPT)], out.at[c, pl.ds(s * RPT, RPT)])


def _make_agg(epc):
    ept = epc // NS
    nsteps = ept // CSA
    rounds = nsteps // NB

    @functools.partial(
        pl.kernel,
        out_type=jax.ShapeDtypeStruct((NC, NP, 128), jnp.float32),
        mesh=_mesh,
        scratch_types=[
            [pltpu.VMEM((CSA,), jnp.int32) for _ in range(NB)],
            [pltpu.VMEM((CSA,), jnp.int32) for _ in range(NB)],
            [pltpu.VMEM((CSA, 128), jnp.float32) for _ in range(NB)],
            pltpu.VMEM_SHARED((NP, 128), jnp.float32),
            [pltpu.SemaphoreType.DMA for _ in range(NB)],
            [pltpu.SemaphoreType.DMA for _ in range(NB)],
            [pltpu.SemaphoreType.DMA for _ in range(NB)],
            [pltpu.SemaphoreType.DMA for _ in range(NB)],
        ],
    )
    def _agg(table, idxs, dsts, zeros, out, idx_v, dst_v, rows_v, acc,
             isem, dsem, gsem, ssem):
        c = lax.axis_index("c")
        s = lax.axis_index("s")
        base = c * epc + s * ept

        for b in range(NB):
            off = base + b * CSA
            pltpu.async_copy(idxs.at[pl.ds(off, CSA)], idx_v[b], isem[b])
            pltpu.async_copy(dsts.at[pl.ds(off, CSA)], dst_v[b], dsem[b])
        for b in range(NB):
            pltpu.make_async_copy(idxs.at[pl.ds(base, CSA)], idx_v[b],
                                  isem[b]).wait()
            pltpu.async_copy(table.at[idx_v[b]], rows_v[b], gsem[b])
        pltpu.sync_copy(zeros.at[pl.ds(s * RPT, RPT)], acc.at[pl.ds(s * RPT, RPT)])
        plsc.subcore_barrier()

        def round_(r, carry):
            for b in range(NB):
                pltpu.make_async_copy(dsts.at[pl.ds(base, CSA)], dst_v[b],
                                      dsem[b]).wait()
                pltpu.make_async_copy(zeros.at[pl.ds(0, CSA)], rows_v[b],
                                      gsem[b]).wait()
                pltpu.async_copy(rows_v[b], acc.at[dst_v[b]], ssem[b], add=True)

                @pl.when(r != rounds - 1)
                def _():
                    off = base + ((r + 1) * NB + b) * CSA
                    pltpu.async_copy(idxs.at[pl.ds(off, CSA)], idx_v[b], isem[b])
            for b in range(NB):
                @pl.when(r != rounds - 1)
                def _():
                    pltpu.make_async_copy(rows_v[b], acc.at[dst_v[b]],
                                          ssem[b]).wait()
                    off = base + ((r + 1) * NB + b) * CSA
                    pltpu.async_copy(dsts.at[pl.ds(off, CSA)], dst_v[b], dsem[b])
                    pltpu.make_async_copy(idxs.at[pl.ds(base, CSA)], idx_v[b],
                                          isem[b]).wait()
                    pltpu.async_copy(table.at[idx_v[b]], rows_v[b], gsem[b])
            return carry

        lax.fori_loop(0, rounds, round_, 0)
        for b in range(NB):
            pltpu.make_async_copy(rows_v[b], acc.at[dst_v[b]], ssem[b]).wait()
        plsc.subcore_barrier()
        pltpu.sync_copy(acc.at[pl.ds(s * RPT, RPT)], out.at[c, pl.ds(s * RPT, RPT)])

    return _agg


_agg_half = _make_agg(E // NC)
_agg_full = _make_agg(E)


def _pre_body(x_ref, l_ref, ptr_ref):
    x = x_ref[...]
    s = jnp.sum(x, axis=1, keepdims=True)
    l_ref[...] = s
    ptr_ref[...] = jnp.log1p(x * (TOTAL_COUNT / s))


def _pre(x):
    return pl.pallas_call(
        _pre_body,
        grid=(N // RB,),
        in_specs=[pl.BlockSpec((RB, D_IN), lambda i: (i, 0))],
        out_specs=[
            pl.BlockSpec((RB, 1), lambda i: (i, 0)),
            pl.BlockSpec((RB, D_IN), lambda i: (i, 0)),
        ],
        out_shape=[
            jax.ShapeDtypeStruct((N, 1), jnp.float32),
            jax.ShapeDtypeStruct((N, D_IN), jnp.float32),
        ],
    )(x)


def _scale_body(degp_ref, ptr_ref, d_ref, t1_ref):
    deg = 1.0 + degp_ref[0, :, 0:1] + degp_ref[1, :, 0:1]
    d = lax.rsqrt(deg)
    d_ref[...] = d
    t1_ref[...] = d * ptr_ref[...]


def _scale(degp, ptr):
    return pl.pallas_call(
        _scale_body,
        grid=(N // RB,),
        in_specs=[
            pl.BlockSpec((2, RB, 16), lambda i: (0, i, 0)),
            pl.BlockSpec((RB, D_IN), lambda i: (i, 0)),
        ],
        out_specs=[
            pl.BlockSpec((RB, 1), lambda i: (i, 0)),
            pl.BlockSpec((RB, D_IN), lambda i: (i, 0)),
        ],
        out_shape=[
            jax.ShapeDtypeStruct((N, 1), jnp.float32),
            jax.ShapeDtypeStruct((N, D_IN), jnp.float32),
        ],
    )(degp, ptr)


def _l1_body(p_ref, t1_ref, d_ref, w_ref, b_ref, out_ref):
    d = d_ref[...]
    agg = d * (p_ref[0] + p_ref[1] + t1_ref[...])
    z = jnp.dot(agg, w_ref[...], preferred_element_type=jnp.float32) + b_ref[...]
    h = jnp.where(z >= 0, z, 0.2 * z)
    out_ref[0] = d * h


def _l1(p, t1, d, w1eff, b1):
    return pl.pallas_call(
        _l1_body,
        grid=(2, N // RB),
        in_specs=[
            pl.BlockSpec((2, RB, 128), lambda h, j: (0, j, 0)),
            pl.BlockSpec((RB, D_IN), lambda h, j: (j, 0)),
            pl.BlockSpec((RB, 1), lambda h, j: (j, 0)),
            pl.BlockSpec((D_IN, 128), lambda h, j: (0, h)),
            pl.BlockSpec((1, 128), lambda h, j: (0, h)),
        ],
        out_specs=pl.BlockSpec((1, RB, 128), lambda h, j: (h, j, 0)),
        out_shape=jax.ShapeDtypeStruct((2, N, 128), jnp.float32),
    )(p, t1, d, w1eff, b1)


def _l2_body(q_ref, t2_ref, d_ref, w2_ref, b2_ref, wl_ref, bl_ref, ws_ref,
             bs_ref, loc_ref, std_ref):
    d = d_ref[...]
    a_lo = d * (q_ref[0] + t2_ref[0])
    a_hi = d * (q_ref[1] + t2_ref[1])
    w2 = w2_ref[...]
    z = (jnp.dot(a_lo, w2[:128], preferred_element_type=jnp.float32)
         + jnp.dot(a_hi, w2[128:], preferred_element_type=jnp.float32)
         + b2_ref[...])
    h = jnp.where(z >= 0, z, 0.2 * z)
    loc_ref[...] = (jnp.dot(h, wl_ref[...], preferred_element_type=jnp.float32)
                    + bl_ref[...])
    t = jnp.dot(h, ws_ref[...], preferred_element_type=jnp.float32) + bs_ref[...]
    std_ref[...] = jnp.maximum(t, 0.0) + jnp.log1p(jnp.exp(-jnp.abs(t))) + EPS


def _l2(q, t2, d, w2eff, b2, wl, bl, ws, bs):
    return pl.pallas_call(
        _l2_body,
        grid=(N // RB,),
        in_specs=[
            pl.BlockSpec((2, RB, 128), lambda j: (0, j, 0)),
            pl.BlockSpec((2, RB, 128), lambda j: (0, j, 0)),
            pl.BlockSpec((RB, 1), lambda j: (j, 0)),
            pl.BlockSpec((H, H), lambda j: (0, 0)),
            pl.BlockSpec((1, H), lambda j: (0, 0)),
            pl.BlockSpec((H, D_OUT), lambda j: (0, 0)),
            pl.BlockSpec((1, D_OUT), lambda j: (0, 0)),
            pl.BlockSpec((H, D_OUT), lambda j: (0, 0)),
            pl.BlockSpec((1, D_OUT), lambda j: (0, 0)),
        ],
        out_specs=[
            pl.BlockSpec((RB, D_OUT), lambda j: (j, 0)),
            pl.BlockSpec((RB, D_OUT), lambda j: (j, 0)),
        ],
        out_shape=[
            jax.ShapeDtypeStruct((N, D_OUT), jnp.float32),
            jax.ShapeDtypeStruct((N, D_OUT), jnp.float32),
        ],
    )(q, t2, d, w2eff, b2, wl, bl, ws, bs)


def kernel(x, edge_index, W1, g1, b1, W2, g2, b2, Wl, bl, Ws, bs):
    src = edge_index[0]
    dst = edge_index[1]

    zeros16 = jnp.zeros((NP, 16), jnp.float32)
    zeros128 = jnp.zeros((NP, 128), jnp.float32)
    ones = jnp.ones((CS, 16), jnp.float32)

    degp = _deg_kernel(dst, ones, zeros16)
    l, ptr = _pre(x)
    d, t1 = _scale(degp, ptr)

    p = _agg_half(t1, src, dst, zeros128)
    bn1 = (1.0 + 1e-5) ** -0.5
    w1eff = W1 * (g1 * bn1)[None, :]
    t2 = _l1(p, t1, d, w1eff, b1.reshape(1, H))

    idx2 = jnp.concatenate([src, src + N])
    dst2 = jnp.concatenate([dst, dst])
    q = _agg_full(t2.reshape(2 * N, 128), idx2, dst2, zeros128)

    w2eff = W2 * (g2 * bn1)[None, :]
    loc, std = _l2(q, t2, d, w2eff, b2.reshape(1, H),
                   Wl, bl.reshape(1, D_OUT), Ws, bs.reshape(1, D_OUT))
    return (loc, std, l)
